# Initial kernel scaffold; baseline (speedup 1.0000x reference)
import functools

import jax
import jax.numpy as jnp
from jax import lax
from jax.experimental import pallas as pl
from jax.experimental.pallas import tpu as pltpu
from jax.experimental.pallas import tpu_sc as plsc

_A = 2048
_D = 128
_B = 512
_S = 256

_NC, _NS, _L = 2, 16, 16
_NW = _NC * _NS
_APW = _A // _NW


def _dist_body(basis_ref, d_ref):
    b = basis_ref[...]
    diff = b - b[0:1, :]
    s = jnp.sum(diff * diff, axis=1)
    d_ref[...] = jnp.sqrt(s).reshape(1, _A)


_dist_call = pl.pallas_call(
    _dist_body,
    out_shape=jax.ShapeDtypeStruct((1, _A), jnp.float32),
)


_sc_mesh = plsc.VectorSubcoreMesh(core_axis_name="c", subcore_axis_name="s")


@functools.partial(
    pl.kernel,
    mesh=_sc_mesh,
    out_type=jax.ShapeDtypeStruct((_A,), jnp.int32),
    scratch_types=[
        pltpu.VMEM((_A,), jnp.float32),
        pltpu.VMEM((_APW,), jnp.int32),
    ],
)
def _sc_rank(d_hbm, rank_hbm, d_v, rank_v):
    wid = lax.axis_index("c") * _NS + lax.axis_index("s")
    base = wid * _APW
    pltpu.sync_copy(d_hbm, d_v)

    for k in range(_APW // _L):
        a_vals = d_v[pl.ds(base + k * _L, _L)]
        a_idx = base + k * _L + lax.iota(jnp.int32, _L)

        def body(j, cnt, a_vals=a_vals, a_idx=a_idx):
            vj = d_v[j]
            m = (vj < a_vals) | ((vj == a_vals) & (j < a_idx))
            return cnt + m.astype(jnp.int32)

        cnt = lax.fori_loop(0, _A, body, jnp.zeros((_L,), jnp.int32),
                            unroll=8)
        rank_v[pl.ds(k * _L, _L)] = cnt

    pltpu.sync_copy(rank_v, rank_hbm.at[pl.ds(base, _APW)])


def _decode_body(rank_ref, x_ref, basis_ref, out_ref):
    rank = rank_ref[...]
    iot = lax.broadcasted_iota(jnp.int32, (_S, _A), 0)
    onehot = jnp.where(iot == rank, 1.0, 0.0)
    bsel = jnp.dot(onehot, basis_ref[...],
                   preferred_element_type=jnp.float32)
    out_ref[...] = jnp.dot(x_ref[...], bsel,
                           preferred_element_type=jnp.float32)


_decode_call = pl.pallas_call(
    _decode_body,
    out_shape=jax.ShapeDtypeStruct((_B, _D), jnp.float32),
)


def kernel(x, basis):
    B, S = x.shape
    A, D = basis.shape
    assert (A, D, B, S) == (_A, _D, _B, _S)
    d = _dist_call(basis)
    rank = _sc_rank(d.reshape(_A))
    out = _decode_call(rank.reshape(1, _A), x, basis)
    return out.reshape(B, D, 1, 1)


# speedup vs baseline: 5.3345x; 5.3345x over previous
import functools

import jax
import jax.numpy as jnp
from jax import lax
from jax.experimental import pallas as pl
from jax.experimental.pallas import tpu as pltpu
from jax.experimental.pallas import tpu_sc as plsc

_A = 2048
_D = 128
_B = 512
_S = 256

_NC, _NS, _L = 2, 16, 16
_NW = _NC * _NS
_APW = _A // _NW


def _dist_body(basis_ref, d_ref):
    b = basis_ref[...]
    diff = b - b[0:1, :]
    s = jnp.sum(diff * diff, axis=1)
    d_ref[...] = jnp.sqrt(s).reshape(1, _A)


_dist_call = pl.pallas_call(
    _dist_body,
    out_shape=jax.ShapeDtypeStruct((1, _A), jnp.float32),
)


_sc_mesh = plsc.VectorSubcoreMesh(core_axis_name="c", subcore_axis_name="s")


@functools.partial(
    pl.kernel,
    mesh=_sc_mesh,
    out_type=jax.ShapeDtypeStruct((_A,), jnp.int32),
    scratch_types=[
        pltpu.VMEM((_A,), jnp.float32),
        pltpu.VMEM((_APW,), jnp.int32),
    ],
)
def _sc_rank(d_hbm, rank_hbm, d_v, rank_v):
    wid = lax.axis_index("c") * _NS + lax.axis_index("s")
    base = wid * _APW
    pltpu.sync_copy(d_hbm, d_v)

    nk = _APW // _L
    a_vals = [d_v[pl.ds(base + k * _L, _L)] for k in range(nk)]
    a_idx = [base + k * _L + lax.iota(jnp.int32, _L) for k in range(nk)]

    def body(j16, cnts):
        dj = d_v[pl.ds(j16 * _L, _L)]
        new = list(cnts)
        for r in range(_L):
            vj = jnp.full((_L,), dj[r], jnp.float32)
            jv = jnp.full((_L,), j16 * _L + r, jnp.int32)
            for k in range(nk):
                m = (vj < a_vals[k]) | ((vj == a_vals[k]) & (jv < a_idx[k]))
                new[k] = new[k] + jnp.where(m, 1, 0)
        return tuple(new)

    cnts = lax.fori_loop(0, _A // _L, body,
                         tuple(jnp.zeros((_L,), jnp.int32) for _ in range(nk)))
    for k in range(nk):
        rank_v[pl.ds(k * _L, _L)] = cnts[k]

    pltpu.sync_copy(rank_v, rank_hbm.at[pl.ds(base, _APW)])


def _decode_body(rank_ref, x_ref, basis_ref, out_ref):
    rank = rank_ref[...]
    iot = lax.broadcasted_iota(jnp.int32, (_S, _A), 0)
    onehot = jnp.where(iot == rank, 1.0, 0.0)
    bsel = jnp.dot(onehot, basis_ref[...],
                   preferred_element_type=jnp.float32)
    out_ref[...] = jnp.dot(x_ref[...], bsel,
                           preferred_element_type=jnp.float32)


_decode_call = pl.pallas_call(
    _decode_body,
    out_shape=jax.ShapeDtypeStruct((_B, _D), jnp.float32),
)


def kernel(x, basis):
    B, S = x.shape
    A, D = basis.shape
    assert (A, D, B, S) == (_A, _D, _B, _S)
    d = _dist_call(basis)
    rank = _sc_rank(d.reshape(_A))
    out = _decode_call(rank.reshape(1, _A), x, basis)
    return out.reshape(B, D, 1, 1)


# speedup vs baseline: 28.2526x; 5.2962x over previous
import functools

import jax
import jax.numpy as jnp
from jax import lax
from jax.experimental import pallas as pl
from jax.experimental.pallas import tpu as pltpu
from jax.experimental.pallas import tpu_sc as plsc

_A = 2048
_D = 128
_B = 512
_S = 256

_NC, _NS, _L = 2, 16, 16
_NW = _NC * _NS
_APW = _A // _NW


_sc_mesh = plsc.VectorSubcoreMesh(core_axis_name="c", subcore_axis_name="s")


_GDN = lax.GatherDimensionNumbers(
    offset_dims=(), collapsed_slice_dims=(0,), start_index_map=(0,))


def _lane_bcast(v, idx_vec):
    return lax.gather(v, idx_vec[:, None], _GDN, (1,),
                      mode=lax.GatherScatterMode.PROMISE_IN_BOUNDS)


@functools.partial(
    pl.kernel,
    mesh=_sc_mesh,
    out_type=jax.ShapeDtypeStruct((_A,), jnp.int32),
    scratch_types=[
        pltpu.VMEM((_A,), jnp.int32),
        pltpu.VMEM((_APW,), jnp.int32),
    ],
)
def _sc_rank(d_hbm, rank_hbm, d_v, rank_v):
    wid = lax.axis_index("c") * _NS + lax.axis_index("s")
    base = wid * _APW
    pltpu.sync_copy(d_hbm, d_v)

    nk = _APW // _L
    iota = lax.iota(jnp.int32, _L)
    bc = [jnp.full((_L,), r, jnp.int32) for r in range(_L)]
    cr = [jnp.full((_L,), r, jnp.int32) - iota for r in range(_L)]
    sixteen = jnp.full((_L,), _L, jnp.int32)
    zero = jnp.zeros((_L,), jnp.int32)
    cs = base // _L

    def make_sweep(thr):
        def body(j16, accs):
            dj = d_v[pl.ds(j16 * _L, _L)]
            accs = list(accs)
            for r in range(_L):
                bj = _lane_bcast(dj, bc[r])
                accs[r % 4] = accs[r % 4] + ((bj - thr) >> 31)
            return tuple(accs)
        return body

    def kbody(kk, kbv):
        bak = d_v[pl.ds(base + kk * _L, _L)]
        ba1k = bak + 1
        accs = (zero, zero, zero, zero)
        accs = lax.fori_loop(0, cs, make_sweep(ba1k), accs)

        def midbody(u, carry):
            accs, ubv = carry
            dj = d_v[pl.ds(base + u * _L, _L)]
            ofs = ubv - kbv
            accs = list(accs)
            for r in range(_L):
                bj = _lane_bcast(dj, bc[r])
                diff = jnp.minimum(jnp.maximum(bj - bak, -1), 1)
                t = (diff << 12) + ofs + cr[r]
                accs[r % 4] = accs[r % 4] + (t >> 31)
            return (tuple(accs), ubv + sixteen)

        accs, _ = lax.fori_loop(0, nk, midbody, (accs, zero))
        accs = lax.fori_loop(cs + nk, _A // _L, make_sweep(bak), accs)
        cnt = (accs[0] + accs[1]) + (accs[2] + accs[3])
        rank_v[pl.ds(kk * _L, _L)] = -cnt
        return kbv + sixteen

    lax.fori_loop(0, nk, kbody, zero)

    pltpu.sync_copy(rank_v, rank_hbm.at[pl.ds(base, _APW)])


def _decode_body(rank_ref, x_ref, basis_ref, out_ref):
    rank = rank_ref[...]
    iot = lax.broadcasted_iota(jnp.int32, (_S, _A), 0)
    onehot = jnp.where(iot == rank, 1.0, 0.0)
    bsel = jnp.dot(onehot, basis_ref[...],
                   preferred_element_type=jnp.float32)
    out_ref[...] = jnp.dot(x_ref[...], bsel,
                           preferred_element_type=jnp.float32)


_decode_call = pl.pallas_call(
    _decode_body,
    out_shape=jax.ShapeDtypeStruct((_B, _D), jnp.float32),
)


def kernel(x, basis):
    B, S = x.shape
    A, D = basis.shape
    assert (A, D, B, S) == (_A, _D, _B, _S)
    d = jnp.sqrt(jnp.sum((basis - basis[0:1, :]) ** 2, axis=1))
    dbits = lax.bitcast_convert_type(d, jnp.int32)
    rank = _sc_rank(dbits)
    out = _decode_call(rank.reshape(1, _A), x, basis)
    return out.reshape(B, D, 1, 1)
